# Initial kernel scaffold; baseline (speedup 1.0000x reference)
import jax
import jax.numpy as jnp
from jax.experimental import pallas as pl


def kernel(x, y, theta):
    raise NotImplementedError("write your pallas kernel here")


# speedup vs baseline: 2.5974x; 2.5974x over previous
import jax
import jax.numpy as jnp
from jax.experimental import pallas as pl
from jax.experimental.pallas import tpu as pltpu


def _cmpex_reshape(v, j, k):
    n, c = v.shape
    g = n // (2 * j)
    v4 = v.reshape(g, 2, j, c)
    a, b = v4[:, 0], v4[:, 1]
    lo, hi = jnp.minimum(a, b), jnp.maximum(a, b)
    gi = jax.lax.broadcasted_iota(jnp.int32, (g, 1, 1), 0)
    asc = ((gi * (2 * j)) & k) == 0
    a2 = jnp.where(asc, lo, hi)
    b2 = jnp.where(asc, hi, lo)
    return jnp.concatenate([a2[:, None], b2[:, None]], axis=1).reshape(n, c)


def _cmpex_roll(v, j, k, row):
    n = v.shape[0]
    down = pltpu.roll(v, n - j, 0)
    up = pltpu.roll(v, j, 0)
    is_a = (row & j) == 0
    partner = jnp.where(is_a, down, up)
    lo, hi = jnp.minimum(v, partner), jnp.maximum(v, partner)
    asc = (row & k) == 0
    keep_lo = is_a == asc
    return jnp.where(keep_lo, lo, hi)


def _bitonic_sort(v):
    n = v.shape[0]
    row = jax.lax.broadcasted_iota(jnp.int32, (n, 1), 0)
    k = 2
    while k <= n:
        j = k // 2
        while j >= 1:
            v = _cmpex_reshape(v, j, k) if j >= 8 else _cmpex_roll(v, j, k, row)
            j //= 2
        k *= 2
    return v


def _swd_body(x_ref, y_ref, tht_ref, s_ref):
    x = x_ref[0]
    y = y_ref[0]
    tht = tht_ref[0]
    xp = jnp.dot(x, tht, preferred_element_type=jnp.float32)
    yp = jnp.dot(y, tht, preferred_element_type=jnp.float32)
    xs = _bitonic_sort(xp)
    ys = _bitonic_sort(yp)
    d = xs - ys
    s_ref[0] = jnp.sum(d * d, axis=0, keepdims=True)


def kernel(x, y, theta):
    b, n, dd = x.shape
    p = theta.shape[1]
    theta_t = theta.transpose(0, 2, 1)
    s = pl.pallas_call(
        _swd_body,
        grid=(b,),
        in_specs=[
            pl.BlockSpec((1, n, dd), lambda i: (i, 0, 0)),
            pl.BlockSpec((1, n, dd), lambda i: (i, 0, 0)),
            pl.BlockSpec((1, dd, p), lambda i: (i, 0, 0)),
        ],
        out_specs=pl.BlockSpec((1, 1, p), lambda i: (i, 0, 0)),
        out_shape=jax.ShapeDtypeStruct((b, 1, p), jnp.float32),
    )(x, y, theta_t)
    distances = jnp.sqrt(jnp.mean(s[:, 0, :], axis=1))
    return jnp.mean(distances)
